# baseline (device time: 12686 ns/iter reference)
import jax
import jax.numpy as jnp
from jax import lax
from jax.experimental import pallas as pl
from jax.experimental.pallas import tpu as pltpu


def kernel(partial, gamma):
    _, m2, d = partial.shape
    m = m2 // 2
    nc = 8
    cm = m // nc

    def body(p_ref, g_ref, o_ref, send_q, recv_q, send_s, recv_s,
             sq_sems, rq_sems, ss_sems, rs_sems):
        my_x = lax.axis_index("x")
        my_y = lax.axis_index("y")
        my_z = lax.axis_index("z")
        partner = (my_x, 1 - my_y, my_z)

        barrier = pltpu.get_barrier_semaphore()
        pl.semaphore_signal(
            barrier, inc=1, device_id=partner,
            device_id_type=pl.DeviceIdType.MESH,
        )
        pl.semaphore_wait(barrier, 1)

        rdmas = []
        for k in range(nc):
            src = p_ref[0, pl.ds((1 - my_y) * m + k * cm, cm), :]
            scale = jnp.max(jnp.abs(src), axis=-1, keepdims=True) / 127.0
            scale = jnp.maximum(scale, 1e-30)
            send_q[k] = jnp.round(src / scale).astype(jnp.int8)
            send_s[k] = scale
            rq = pltpu.make_async_remote_copy(
                src_ref=send_q.at[k],
                dst_ref=recv_q.at[k],
                send_sem=sq_sems.at[k],
                recv_sem=rq_sems.at[k],
                device_id=partner,
                device_id_type=pl.DeviceIdType.MESH,
            )
            rs = pltpu.make_async_remote_copy(
                src_ref=send_s.at[k],
                dst_ref=recv_s.at[k],
                send_sem=ss_sems.at[k],
                recv_sem=rs_sems.at[k],
                device_id=partner,
                device_id_type=pl.DeviceIdType.MESH,
            )
            rq.start()
            rs.start()
            rdmas.append((rq, rs))

        for k in range(nc):
            rdmas[k][0].wait_recv()
            rdmas[k][1].wait_recv()
            contrib = recv_q[k].astype(jnp.float32) * recv_s[k]
            y = p_ref[0, pl.ds(my_y * m + k * cm, cm), :] + contrib
            rms = jnp.sqrt(jnp.mean(y * y, axis=-1, keepdims=True) + 1e-6)
            o_ref[pl.ds(k * cm, cm), :] = (y / rms * g_ref[...]).astype(
                jnp.bfloat16
            )

        for k in range(nc):
            rdmas[k][0].wait_send()
            rdmas[k][1].wait_send()

    return pl.pallas_call(
        body,
        out_shape=jax.ShapeDtypeStruct((m, d), jnp.bfloat16),
        in_specs=[
            pl.BlockSpec(memory_space=pltpu.VMEM),
            pl.BlockSpec(memory_space=pltpu.VMEM),
        ],
        out_specs=pl.BlockSpec(memory_space=pltpu.VMEM),
        scratch_shapes=[
            pltpu.VMEM((nc, cm, d), jnp.int8),
            pltpu.VMEM((nc, cm, d), jnp.int8),
            pltpu.VMEM((nc, cm, 1), jnp.float32),
            pltpu.VMEM((nc, cm, 1), jnp.float32),
            pltpu.SemaphoreType.DMA((nc,)),
            pltpu.SemaphoreType.DMA((nc,)),
            pltpu.SemaphoreType.DMA((nc,)),
            pltpu.SemaphoreType.DMA((nc,)),
        ],
        compiler_params=pltpu.CompilerParams(collective_id=0),
    )(partial, gamma)


# device time: 12378 ns/iter; 1.0249x vs baseline; 1.0249x over previous
import jax
import jax.numpy as jnp
from jax import lax
from jax.experimental import pallas as pl
from jax.experimental.pallas import tpu as pltpu


def kernel(partial, gamma):
    _, m2, d = partial.shape
    m = m2 // 2
    nc = 8
    cm = m // nc

    def body(p_ref, g_ref, o_ref, send_buf, recv_buf, send_sems, recv_sems):
        my_x = lax.axis_index("x")
        my_y = lax.axis_index("y")
        my_z = lax.axis_index("z")
        partner = (my_x, 1 - my_y, my_z)

        barrier = pltpu.get_barrier_semaphore()
        pl.semaphore_signal(
            barrier, inc=1, device_id=partner,
            device_id_type=pl.DeviceIdType.MESH,
        )
        pl.semaphore_wait(barrier, 1)

        rdmas = []
        for k in range(nc):
            send_buf[k] = p_ref[
                0, pl.ds((1 - my_y) * m + k * cm, cm), :
            ].astype(jnp.bfloat16)
            rdma = pltpu.make_async_remote_copy(
                src_ref=send_buf.at[k],
                dst_ref=recv_buf.at[k],
                send_sem=send_sems.at[k],
                recv_sem=recv_sems.at[k],
                device_id=partner,
                device_id_type=pl.DeviceIdType.MESH,
            )
            rdma.start()
            rdmas.append(rdma)

        for k in range(nc):
            rdmas[k].wait_recv()
            y = p_ref[0, pl.ds(my_y * m + k * cm, cm), :] + recv_buf[
                k
            ].astype(jnp.float32)
            rms = jnp.sqrt(jnp.mean(y * y, axis=-1, keepdims=True) + 1e-6)
            o_ref[pl.ds(k * cm, cm), :] = (y / rms * g_ref[...]).astype(
                jnp.bfloat16
            )

        for k in range(nc):
            rdmas[k].wait_send()

    return pl.pallas_call(
        body,
        out_shape=jax.ShapeDtypeStruct((m, d), jnp.bfloat16),
        in_specs=[
            pl.BlockSpec(memory_space=pltpu.VMEM),
            pl.BlockSpec(memory_space=pltpu.VMEM),
        ],
        out_specs=pl.BlockSpec(memory_space=pltpu.VMEM),
        scratch_shapes=[
            pltpu.VMEM((nc, cm, d), jnp.bfloat16),
            pltpu.VMEM((nc, cm, d), jnp.bfloat16),
            pltpu.SemaphoreType.DMA((nc,)),
            pltpu.SemaphoreType.DMA((nc,)),
        ],
        compiler_params=pltpu.CompilerParams(collective_id=0),
    )(partial, gamma)
